# baseline (speedup 1.0000x reference)
import functools

import jax
import jax.numpy as jnp
from jax.experimental import pallas as pl
from jax.experimental.pallas import tpu as pltpu

C = 1.0
MIN_NORM = 1e-15
EPS = 4e-3


def _hypagg_kernel(x_ref, adj_ref, o_ref, xt_ref, acc_ref, *, n, bk, nk):
    i, k = pl.program_id(0), pl.program_id(1)
    rem = n - (nk - 1) * bk

    @pl.when(i == 0)
    def _tangent():
        x = x_ref[...]
        rows = jax.lax.broadcasted_iota(jnp.int32, x.shape, 0)
        x = jnp.where((k < nk - 1) | (rows < rem), x, 0.0)
        nrm = jnp.maximum(
            jnp.sqrt(jnp.sum(x * x, axis=-1, keepdims=True)), MIN_NORM
        )
        t = jnp.clip(nrm, -1.0 + 1e-7, 1.0 - 1e-7)
        at = 0.5 * (jnp.log1p(t) - jnp.log1p(-t))
        xt_ref[pl.ds(k * bk, bk), :] = x / nrm * at

    a = adj_ref[...]
    cols = jax.lax.broadcasted_iota(jnp.int32, a.shape, 1)
    a = jnp.where((k < nk - 1) | (cols < rem), a, 0.0)
    u = jax.lax.dot_general(
        a, xt_ref[pl.ds(k * bk, bk), :], (((1,), (0,)), ((), ())),
        preferred_element_type=jnp.float32,
        precision=jax.lax.Precision.DEFAULT,
    )

    @pl.when(k == 0)
    def _init():
        acc_ref[...] = u

    @pl.when(k > 0)
    def _accum():
        acc_ref[...] += u

    @pl.when(k == nk - 1)
    def _epilogue():
        s = acc_ref[...]
        un = jnp.maximum(
            jnp.sqrt(jnp.sum(s * s, axis=-1, keepdims=True)), MIN_NORM
        )
        y = jnp.tanh(un) * s / un
        yn = jnp.maximum(
            jnp.sqrt(jnp.sum(y * y, axis=-1, keepdims=True)), MIN_NORM
        )
        maxnorm = 1.0 - EPS
        o_ref[...] = jnp.where(yn > maxnorm, y / yn * maxnorm, y)


def _pick_block(n, candidates):
    for c in candidates:
        if n % c == 0 and c % 8 == 0:
            return c
    return n


def kernel(x, adj):
    n, d = x.shape
    bm = _pick_block(n, (400, 512, 256, 200, 128, 80, 64, 40, 16, 8))
    bk = 1024
    nk = -(-n // bk)
    nkp = nk * bk

    out = pl.pallas_call(
        functools.partial(_hypagg_kernel, n=n, bk=bk, nk=nk),
        grid=(n // bm, nk),
        in_specs=[
            pl.BlockSpec((bk, d), lambda i, k: (k, 0)),
            pl.BlockSpec((bm, bk), lambda i, k: (i, k)),
        ],
        out_specs=pl.BlockSpec((bm, d), lambda i, k: (i, 0)),
        out_shape=jax.ShapeDtypeStruct((n, d), jnp.float32),
        scratch_shapes=[
            pltpu.VMEM((nkp, d), jnp.float32),
            pltpu.VMEM((bm, d), jnp.float32),
        ],
        compiler_params=pltpu.CompilerParams(
            dimension_semantics=("arbitrary", "arbitrary"),
        ),
    )(x, adj)
    return out


# speedup vs baseline: 2.1326x; 2.1326x over previous
import jax
import jax.numpy as jnp
from jax.experimental import pallas as pl
from jax.experimental.pallas import tpu as pltpu

C = 1.0
MIN_NORM = 1e-15
EPS = 4e-3


def _hypagg_kernel(x_ref, adj_ref, o_ref, xt_ref):
    @pl.when(pl.program_id(0) == 0)
    def _tangent():
        x = x_ref[...]
        nrm = jnp.maximum(
            jnp.sqrt(jnp.sum(x * x, axis=-1, keepdims=True)), MIN_NORM
        )
        t = jnp.clip(nrm, -1.0 + 1e-7, 1.0 - 1e-7)
        at = 0.5 * (jnp.log1p(t) - jnp.log1p(-t))
        xt_ref[...] = x / nrm * at

    u = jax.lax.dot_general(
        adj_ref[...], xt_ref[...], (((1,), (0,)), ((), ())),
        preferred_element_type=jnp.float32,
        precision=jax.lax.Precision.DEFAULT,
    )
    un = jnp.maximum(
        jnp.sqrt(jnp.sum(u * u, axis=-1, keepdims=True)), MIN_NORM
    )
    y = jnp.tanh(un) * u / un
    yn = jnp.maximum(
        jnp.sqrt(jnp.sum(y * y, axis=-1, keepdims=True)), MIN_NORM
    )
    maxnorm = 1.0 - EPS
    o_ref[...] = jnp.where(yn > maxnorm, y / yn * maxnorm, y)


def _pick_block(n, candidates):
    for c in candidates:
        if n % c == 0 and c % 8 == 0:
            return c
    return n


def kernel(x, adj):
    n, d = x.shape
    bm = _pick_block(n, (400, 512, 256, 200, 128, 80, 64, 40, 16, 8))

    out = pl.pallas_call(
        _hypagg_kernel,
        grid=(n // bm,),
        in_specs=[
            pl.BlockSpec((n, d), lambda i: (0, 0)),
            pl.BlockSpec((bm, n), lambda i: (i, 0)),
        ],
        out_specs=pl.BlockSpec((bm, d), lambda i: (i, 0)),
        out_shape=jax.ShapeDtypeStruct((n, d), jnp.float32),
        scratch_shapes=[pltpu.VMEM((n, d), jnp.float32)],
        compiler_params=pltpu.CompilerParams(
            dimension_semantics=("arbitrary",),
        ),
    )(x, adj)
    return out
